# Initial kernel scaffold; baseline (speedup 1.0000x reference)
import jax
import jax.numpy as jnp
from jax.experimental import pallas as pl


def kernel(xyz, points, params):
    raise NotImplementedError("write your pallas kernel here")


# speedup vs baseline: 1.0244x; 1.0244x over previous
import jax
import jax.numpy as jnp
from jax.experimental import pallas as pl

_NPOINT = 512
_RADIUS_LIST = [0.1, 0.2, 0.4]
_NSAMPLE_LIST = [16, 32, 128]
_MLP_LIST = [[32, 32, 64], [64, 64, 128], [64, 96, 128]]
_TM = 4096


def _matmul_stats_kernel(x_ref, w_ref, b_ref, y_ref, stats_ref):
    i = pl.program_id(0)
    y = jnp.dot(w_ref[...], x_ref[...], preferred_element_type=jnp.float32)
    y = y + b_ref[...]
    y_ref[...] = y

    @pl.when(i == 0)
    def _():
        stats_ref[...] = jnp.zeros_like(stats_ref)

    s = jnp.sum(y, axis=1)
    ss = jnp.sum(y * y, axis=1)
    stats_ref[...] += jnp.stack([s, ss], axis=0)


def _scale_relu_kernel(y_ref, a_ref, c_ref, o_ref):
    o_ref[...] = jnp.maximum(y_ref[...] * a_ref[...] + c_ref[...], 0.0)


def _conv_bn_relu(x2, W, b, g, be):
    O, C = W.shape
    M = x2.shape[1]
    grid = (M // _TM,)
    y, stats = pl.pallas_call(
        _matmul_stats_kernel,
        grid=grid,
        in_specs=[
            pl.BlockSpec((C, _TM), lambda i: (0, i)),
            pl.BlockSpec((O, C), lambda i: (0, 0)),
            pl.BlockSpec((O, 1), lambda i: (0, 0)),
        ],
        out_specs=[
            pl.BlockSpec((O, _TM), lambda i: (0, i)),
            pl.BlockSpec((2, O), lambda i: (0, 0)),
        ],
        out_shape=[
            jax.ShapeDtypeStruct((O, M), jnp.float32),
            jax.ShapeDtypeStruct((2, O), jnp.float32),
        ],
    )(x2, W, b.reshape(O, 1))
    mean = stats[0] / M
    var = stats[1] / M - mean * mean
    a = g * jax.lax.rsqrt(var + 1e-5)
    c = be - a * mean
    out = pl.pallas_call(
        _scale_relu_kernel,
        grid=grid,
        in_specs=[
            pl.BlockSpec((O, _TM), lambda i: (0, i)),
            pl.BlockSpec((O, 1), lambda i: (0, 0)),
            pl.BlockSpec((O, 1), lambda i: (0, 0)),
        ],
        out_specs=pl.BlockSpec((O, _TM), lambda i: (0, i)),
        out_shape=jax.ShapeDtypeStruct((O, M), jnp.float32),
    )(y, a.reshape(O, 1), c.reshape(O, 1))
    return out


def _square_distance(src, dst):
    B, N, _ = src.shape
    M = dst.shape[1]
    dist = -2.0 * jnp.matmul(src, jnp.transpose(dst, (0, 2, 1)))
    dist = dist + jnp.sum(src ** 2, -1).reshape(B, N, 1)
    dist = dist + jnp.sum(dst ** 2, -1).reshape(B, 1, M)
    return dist


def _index_points(points, idx):
    return jax.vmap(lambda p, i: p[i])(points, idx)


def _farthest_point_sample(xyz, npoint):
    B, N, _ = xyz.shape
    centroids0 = jnp.zeros((B, npoint), dtype=jnp.int32)
    distance0 = jnp.full((B, N), 1e10, dtype=xyz.dtype)
    farthest0 = jnp.zeros((B,), dtype=jnp.int32)

    def body(i, state):
        centroids, distance, farthest = state
        centroids = centroids.at[:, i].set(farthest)
        centroid = xyz[jnp.arange(B), farthest, :].reshape(B, 1, 3)
        dist = jnp.sum((xyz - centroid) ** 2, -1)
        distance = jnp.minimum(distance, dist)
        farthest = jnp.argmax(distance, axis=-1).astype(jnp.int32)
        return (centroids, distance, farthest)

    centroids, _, _ = jax.lax.fori_loop(0, npoint, body,
                                        (centroids0, distance0, farthest0))
    return centroids


def _query_ball_point(radius, nsample, xyz, new_xyz):
    B, N, _ = xyz.shape
    S = new_xyz.shape[1]
    sqrdists = _square_distance(new_xyz, xyz)
    group_idx = jnp.broadcast_to(jnp.arange(N, dtype=jnp.int32), (B, S, N))
    group_idx = jnp.where(sqrdists > radius ** 2, N, group_idx)
    group_idx = jnp.sort(group_idx, axis=-1)[:, :, :nsample]
    group_first = jnp.broadcast_to(group_idx[:, :, 0:1], (B, S, nsample))
    group_idx = jnp.where(group_idx == N, group_first, group_idx)
    return group_idx


def kernel(xyz, points, params):
    xyz_t = jnp.transpose(xyz, (0, 2, 1))
    pts = jnp.transpose(points, (0, 2, 1))
    B, N, C = xyz_t.shape
    S = _NPOINT
    fps_idx = _farthest_point_sample(xyz_t, S)
    new_xyz = _index_points(xyz_t, fps_idx)
    new_points_list = []
    for i, radius in enumerate(_RADIUS_LIST):
        K = _NSAMPLE_LIST[i]
        group_idx = _query_ball_point(radius, K, xyz_t, new_xyz)
        grouped_xyz = _index_points(xyz_t, group_idx)
        grouped_xyz = grouped_xyz - new_xyz.reshape(B, S, 1, C)
        grouped_points = _index_points(pts, group_idx)
        grouped_points = jnp.concatenate([grouped_points, grouped_xyz], axis=-1)
        x = jnp.transpose(grouped_points, (3, 0, 2, 1))
        Cin = x.shape[0]
        x2 = x.reshape(Cin, B * K * S)
        for j in range(len(_MLP_LIST[i])):
            x2 = _conv_bn_relu(x2, params[f"W_{i}_{j}"], params[f"b_{i}_{j}"],
                               params[f"g_{i}_{j}"], params[f"be_{i}_{j}"])
        Cout = x2.shape[0]
        y = x2.reshape(Cout, B, K, S)
        new_points = jnp.max(y, axis=2)
        new_points_list.append(jnp.transpose(new_points, (1, 0, 2)))
    new_xyz_out = jnp.transpose(new_xyz, (0, 2, 1))
    new_points_concat = jnp.concatenate(new_points_list, axis=1)
    return (new_xyz_out, new_points_concat)
